# baseline (speedup 1.0000x reference)
import jax
import jax.numpy as jnp
import numpy as np
from jax.experimental import pallas as pl
from jax.experimental.pallas import tpu as pltpu
from jax.experimental.pallas import tpu_sc as plsc

B, S, H = 16, 38, 768
IMG = 512
D = H + IMG * 2
NH = 4
DK = D // NH
NP = 64
NQ = B * S
NW = 32
NCHUNK = NQ // 4
CPW = (NCHUNK + NW - 1) // NW

_HI = jax.lax.Precision.HIGHEST


def _dot(a, b, dims=((1,), (0,))):
    return jax.lax.dot_general(a, b, (dims, ((), ())), precision=_HI,
                               preferred_element_type=jnp.float32)


def _prep_kernel(proto_ref, wq_ref, bq_ref, wk_ref, bk_ref, wv_ref, bv_ref,
                 t_ref, sb_ref, v0_ref):
    p = proto_ref[...]
    k0 = _dot(p, wk_ref[...]) + bk_ref[...]
    v0_ref[...] = _dot(p, wv_ref[...]) + bv_ref[...]
    for h in range(NH):
        hs = slice(h * DK, (h + 1) * DK)
        k0h = k0[:, hs]
        t_ref[:, h * NP:(h + 1) * NP] = _dot(wq_ref[:, hs], k0h,
                                             ((1,), (1,)))
        sb_ref[:, h * NP:(h + 1) * NP] = _dot(bq_ref[:, hs], k0h,
                                              ((1,), (1,)))


def _prep_w0_kernel(v0_ref, wo0_ref, wo1_ref, wo2_ref, wo3_ref, wfcb_ref,
                    bo_ref, bfc_ref, wqap_ref, z0_ref, vb_ref):
    wfcb = wfcb_ref[...]
    wqap = wqap_ref[...]
    wo_refs = (wo0_ref, wo1_ref, wo2_ref, wo3_ref)
    for h in range(NH):
        hs = slice(h * DK, (h + 1) * DK)
        u0h = _dot(v0_ref[:, hs], wo_refs[h][...])
        w0h = _dot(u0h, wfcb)
        z0_ref[h * NP:(h + 1) * NP, :] = _dot(w0h, wqap)
    vb_ref[...] = _dot(bo_ref[...], wfcb) + bfc_ref[...]


def _scores_kernel(cf_ref, t_ref, sb_ref, wfct_ref, scores_ref, cfw_ref):
    cf = cf_ref[...]
    scores_ref[...] = _dot(cf, t_ref[...]) + sb_ref[...]
    cfw_ref[...] = _dot(cf, wfct_ref[...])


_NGRP = NQ // 16
_GPW = (_NGRP + NW - 1) // NW


def _sc_zsel_body(scores_hbm, z0_hbm, out_hbm, st_v, rows_v, acc_v, dsem):
    c = jax.lax.axis_index("c")
    s = jax.lax.axis_index("s")
    wid = s * 2 + c
    lanes = jax.lax.iota(jnp.int32, 16)

    def col(j):
        return plsc.load_gather(st_v, [lanes, jnp.full((16,), j, jnp.int32)])

    def grp_body(i, carry):
        g = wid + NW * i

        @pl.when(g < _NGRP)
        def _():
            base = g * 16
            pltpu.sync_copy(scores_hbm.at[pl.ds(base, 16)], st_v)
            copies = []
            for h in range(NH):
                m = col(h * NP)
                idx = jnp.zeros((16,), jnp.int32)
                for j in range(1, NP):
                    v = col(h * NP + j)
                    gt = v > m
                    m = jnp.where(gt, v, m)
                    idx = jnp.where(gt, j, idx)
                copies.append(pltpu.async_copy(
                    z0_hbm.at[idx + h * NP],
                    rows_v.at[pl.ds(h * 16, 16)], dsem))
            for cp in copies:
                cp.wait()
            for r in range(16):
                sl = pl.ds(0, 16)
                acc_v[r, :] = (rows_v[r, sl] + rows_v[16 + r, sl]
                               + rows_v[32 + r, sl] + rows_v[48 + r, sl])
            pltpu.sync_copy(acc_v, out_hbm.at[pl.ds(base, 16)])
        return carry

    jax.lax.fori_loop(0, _GPW, grp_body, 0)


def _sc_zsel(scores, z0):
    mesh = plsc.VectorSubcoreMesh(core_axis_name="c", subcore_axis_name="s")
    run = pl.kernel(
        _sc_zsel_body,
        out_type=jax.ShapeDtypeStruct((NQ, 16), jnp.float32),
        mesh=mesh,
        compiler_params=pltpu.CompilerParams(needs_layout_passes=False),
        scratch_types=[
            pltpu.VMEM((16, NH * NP), jnp.float32),
            pltpu.VMEM((NH * 16, 128), jnp.float32),
            pltpu.VMEM((16, 16), jnp.float32),
            pltpu.SemaphoreType.DMA,
        ],
    )
    return run(scores, z0)


def _tail_kernel(cfw_ref, zsel_ref, vb_ref, wqa_ref, bqa_ref,
                 spos_ref, epos_ref, slog_ref, elog_ref, loss_ref):
    reduced = cfw_ref[...] + vb_ref[...]
    logits = (_dot(reduced, wqa_ref[...]) + zsel_ref[:, 0:2]
              + bqa_ref[...])
    slog_ref[...] = logits[:, 0:1]
    elog_ref[...] = logits[:, 1:2]

    rowi = jax.lax.broadcasted_iota(jnp.int32, (B, NQ), 1)
    bi = jax.lax.broadcasted_iota(jnp.int32, (B, NQ), 0)
    onb = ((rowi >= bi * S) & (rowi < (bi + 1) * S)).astype(jnp.float32)
    sums = _dot(onb, jnp.exp(logits))
    lse = jnp.log(sums)
    ons = (rowi == bi * S + spos_ref[...]).astype(jnp.float32)
    one = (rowi == bi * S + epos_ref[...]).astype(jnp.float32)
    sel_s = _dot(ons, logits[:, 0:1])
    sel_e = _dot(one, logits[:, 1:2])
    loss = 0.5 * (jnp.mean(lse[:, 0:1] - sel_s) +
                  jnp.mean(lse[:, 1:2] - sel_e))
    loss_ref[...] = jnp.reshape(loss, (1, 1))


def _f32(shape):
    return jax.ShapeDtypeStruct(shape, jnp.float32)


def kernel(combined_features, attention_mask, start_positions, end_positions,
           prototype_vectors, Wq, bq, Wk, bk, Wv, bv, Wo, bo, Wfc, bfc,
           Wqa, bqa):
    cf2d = combined_features.reshape(NQ, D)
    row = lambda x: x.reshape(1, -1)

    t, sbias, v0 = pl.pallas_call(
        _prep_kernel,
        out_shape=(_f32((D, NH * NP)), _f32((1, NH * NP)), _f32((NP, D))),
    )(prototype_vectors, Wq, row(bq), Wk, row(bk), Wv, row(bv))

    spos = start_positions.astype(jnp.int32).reshape(B, 1)
    epos = end_positions.astype(jnp.int32).reshape(B, 1)
    wqa_pad = jnp.pad(Wqa, ((0, 0), (0, 126)))

    z0, vbias = pl.pallas_call(
        _prep_w0_kernel,
        out_shape=(_f32((NH * NP, 128)), _f32((1, H))),
    )(v0,
      Wo[0 * DK:1 * DK, :], Wo[1 * DK:2 * DK, :],
      Wo[2 * DK:3 * DK, :], Wo[3 * DK:4 * DK, :],
      Wfc[D:, :], row(bo), row(bfc), wqa_pad)

    scores, cfw = pl.pallas_call(
        _scores_kernel,
        out_shape=(_f32((NQ, NH * NP)), _f32((NQ, H))),
    )(cf2d, t, sbias, Wfc[:D, :])

    zsel = _sc_zsel(scores, z0)

    slog, elog, loss = pl.pallas_call(
        _tail_kernel,
        out_shape=(_f32((NQ, 1)), _f32((NQ, 1)), _f32((1, 1))),
    )(cfw, zsel, vbias, Wqa, row(bqa), spos, epos)

    return loss[0, 0], slog.reshape(B, S), elog.reshape(B, S)


# speedup vs baseline: 1.0758x; 1.0758x over previous
import jax
import jax.numpy as jnp
import numpy as np
from jax.experimental import pallas as pl
from jax.experimental.pallas import tpu as pltpu
from jax.experimental.pallas import tpu_sc as plsc

B, S, H = 16, 38, 768
IMG = 512
D = H + IMG * 2
NH = 4
DK = D // NH
NP = 64
NQ = B * S
NW = 32
NCHUNK = NQ // 4
CPW = (NCHUNK + NW - 1) // NW

_HI = jax.lax.Precision.HIGHEST


def _dot(a, b, dims=((1,), (0,))):
    return jax.lax.dot_general(a, b, (dims, ((), ())), precision=_HI,
                               preferred_element_type=jnp.float32)


def _prep_kernel(proto_ref, wq_ref, bq_ref, wk_ref, bk_ref, wv_ref, bv_ref,
                 t_ref, sb_ref, v0_ref):
    p = proto_ref[...]
    k0 = _dot(p, wk_ref[...]) + bk_ref[...]
    v0_ref[...] = _dot(p, wv_ref[...]) + bv_ref[...]
    for h in range(NH):
        hs = slice(h * DK, (h + 1) * DK)
        k0h = k0[:, hs]
        t_ref[:, h * NP:(h + 1) * NP] = _dot(wq_ref[:, hs], k0h,
                                             ((1,), (1,)))
        sb_ref[:, h * NP:(h + 1) * NP] = _dot(bq_ref[:, hs], k0h,
                                              ((1,), (1,)))


def _prep_w0_kernel(v0_ref, wo0_ref, wo1_ref, wo2_ref, wo3_ref, wfcb_ref,
                    bo_ref, bfc_ref, wqap_ref, z0_ref, vb_ref):
    wfcb = wfcb_ref[...]
    wqap = wqap_ref[...]
    wo_refs = (wo0_ref, wo1_ref, wo2_ref, wo3_ref)
    for h in range(NH):
        hs = slice(h * DK, (h + 1) * DK)
        u0h = _dot(v0_ref[:, hs], wo_refs[h][...])
        w0h = _dot(u0h, wfcb)
        z0_ref[h * NP:(h + 1) * NP, :] = _dot(w0h, wqap)
    vb_ref[...] = _dot(bo_ref[...], wfcb) + bfc_ref[...]


def _scores_kernel(cf_ref, t_ref, sb_ref, scores_ref):
    scores_ref[...] = _dot(cf_ref[...], t_ref[...]) + sb_ref[...]


def _cfw_kernel(cf_ref, wfct_ref, cfw_ref):
    cfw_ref[...] = _dot(cf_ref[...], wfct_ref[...])


_NGRP = NQ // 16
_GPW = (_NGRP + NW - 1) // NW


def _sc_zsel_body(scores_hbm, z0_hbm, out_hbm, st_v, rows_v, acc_v, dsem):
    c = jax.lax.axis_index("c")
    s = jax.lax.axis_index("s")
    wid = s * 2 + c
    lanes = jax.lax.iota(jnp.int32, 16)

    def col(j):
        return plsc.load_gather(st_v, [lanes, jnp.full((16,), j, jnp.int32)])

    def grp_body(i, carry):
        g = wid + NW * i

        @pl.when(g < _NGRP)
        def _():
            base = g * 16
            pltpu.sync_copy(scores_hbm.at[pl.ds(base, 16)], st_v)
            copies = []
            for h in range(NH):
                m = col(h * NP)
                idx = jnp.zeros((16,), jnp.int32)
                for j in range(1, NP):
                    v = col(h * NP + j)
                    gt = v > m
                    m = jnp.where(gt, v, m)
                    idx = jnp.where(gt, j, idx)
                copies.append(pltpu.async_copy(
                    z0_hbm.at[idx + h * NP],
                    rows_v.at[pl.ds(h * 16, 16)], dsem))
            for cp in copies:
                cp.wait()
            for r in range(16):
                sl = pl.ds(0, 16)
                acc_v[r, :] = (rows_v[r, sl] + rows_v[16 + r, sl]
                               + rows_v[32 + r, sl] + rows_v[48 + r, sl])
            pltpu.sync_copy(acc_v, out_hbm.at[pl.ds(base, 16)])
        return carry

    jax.lax.fori_loop(0, _GPW, grp_body, 0)


def _sc_zsel(scores, z0):
    mesh = plsc.VectorSubcoreMesh(core_axis_name="c", subcore_axis_name="s")
    run = pl.kernel(
        _sc_zsel_body,
        out_type=jax.ShapeDtypeStruct((NQ, 16), jnp.float32),
        mesh=mesh,
        compiler_params=pltpu.CompilerParams(needs_layout_passes=False),
        scratch_types=[
            pltpu.VMEM((16, NH * NP), jnp.float32),
            pltpu.VMEM((NH * 16, 128), jnp.float32),
            pltpu.VMEM((16, 16), jnp.float32),
            pltpu.SemaphoreType.DMA,
        ],
    )
    return run(scores, z0)


def _tail_kernel(cfw_ref, zsel_ref, vb_ref, wqa_ref, bqa_ref,
                 spos_ref, epos_ref, slog_ref, elog_ref, loss_ref):
    reduced = cfw_ref[...] + vb_ref[...]
    logits = (_dot(reduced, wqa_ref[...]) + zsel_ref[:, 0:2]
              + bqa_ref[...])
    slog_ref[...] = logits[:, 0:1]
    elog_ref[...] = logits[:, 1:2]

    rowi = jax.lax.broadcasted_iota(jnp.int32, (B, NQ), 1)
    bi = jax.lax.broadcasted_iota(jnp.int32, (B, NQ), 0)
    onb = ((rowi >= bi * S) & (rowi < (bi + 1) * S)).astype(jnp.float32)
    sums = _dot(onb, jnp.exp(logits))
    lse = jnp.log(sums)
    ons = (rowi == bi * S + spos_ref[...]).astype(jnp.float32)
    one = (rowi == bi * S + epos_ref[...]).astype(jnp.float32)
    sel_s = _dot(ons, logits[:, 0:1])
    sel_e = _dot(one, logits[:, 1:2])
    loss = 0.5 * (jnp.mean(lse[:, 0:1] - sel_s) +
                  jnp.mean(lse[:, 1:2] - sel_e))
    loss_ref[...] = jnp.reshape(loss, (1, 1))


def _f32(shape):
    return jax.ShapeDtypeStruct(shape, jnp.float32)


def kernel(combined_features, attention_mask, start_positions, end_positions,
           prototype_vectors, Wq, bq, Wk, bk, Wv, bv, Wo, bo, Wfc, bfc,
           Wqa, bqa):
    cf2d = combined_features.reshape(NQ, D)
    row = lambda x: x.reshape(1, -1)

    t, sbias, v0 = pl.pallas_call(
        _prep_kernel,
        out_shape=(_f32((D, NH * NP)), _f32((1, NH * NP)), _f32((NP, D))),
    )(prototype_vectors, Wq, row(bq), Wk, row(bk), Wv, row(bv))

    spos = start_positions.astype(jnp.int32).reshape(B, 1)
    epos = end_positions.astype(jnp.int32).reshape(B, 1)
    wqa_pad = jnp.pad(Wqa, ((0, 0), (0, 126)))

    z0, vbias = pl.pallas_call(
        _prep_w0_kernel,
        out_shape=(_f32((NH * NP, 128)), _f32((1, H))),
    )(v0,
      Wo[0 * DK:1 * DK, :], Wo[1 * DK:2 * DK, :],
      Wo[2 * DK:3 * DK, :], Wo[3 * DK:4 * DK, :],
      Wfc[D:, :], row(bo), row(bfc), wqa_pad)

    scores = pl.pallas_call(
        _scores_kernel,
        out_shape=_f32((NQ, NH * NP)),
    )(cf2d, t, sbias)

    zsel = _sc_zsel(scores, z0)

    cfw = pl.pallas_call(
        _cfw_kernel,
        out_shape=_f32((NQ, H)),
    )(cf2d, Wfc[:D, :])

    slog, elog, loss = pl.pallas_call(
        _tail_kernel,
        out_shape=(_f32((NQ, 1)), _f32((NQ, 1)), _f32((1, 1))),
    )(cfw, zsel, vbias, Wqa, row(bqa), spos, epos)

    return loss[0, 0], slog.reshape(B, S), elog.reshape(B, S)
